# Initial kernel scaffold; baseline (speedup 1.0000x reference)
import jax
import jax.numpy as jnp
from jax.experimental import pallas as pl


def kernel(x_t, edge_index_t, edge_type_t, x_t_batch, x_p, edge_index_p, edge_type_p, x_p_batch, t_Wr1, t_Wo1, t_bb1, t_g1, t_be1, t_Wr2, t_Wo2, t_bb2, t_g2, t_be2, p_Wr1, p_Wo1, p_bb1, p_g1, p_be1, p_Wr2, p_Wo2, p_bb2, p_g2, p_be2):
    raise NotImplementedError("write your pallas kernel here")


# speedup vs baseline: 2.8134x; 2.8134x over previous
import functools

import jax
import jax.numpy as jnp
from jax import lax
from jax.experimental import pallas as pl
from jax.experimental.pallas import tpu as pltpu
from jax.experimental.pallas import tpu_sc as plsc

N = 10000
E = 160000
D = 128
NB = 64
NREL = 2
NSUB = 16
LANES = 16

EPC = E // NSUB
NVR = EPC // LANES
KCH = 128
NCHUNK = (EPC + KCH - 1) // KCH
NPAD = 10240
TRASH = 10016
RPS = NPAD // NSUB
CW = 128

@functools.cache
def _make_edge_prep():
    mesh = plsc.VectorSubcoreMesh(core_axis_name="c", subcore_axis_name="s")
    return functools.partial(
        pl.kernel,
        mesh=mesh,
        out_type=[
            jax.ShapeDtypeStruct((NREL, NSUB, NCHUNK, KCH), jnp.int32),
            jax.ShapeDtypeStruct((NREL, NSUB, NCHUNK, KCH), jnp.int32),
        ],
        scratch_types=[
            pltpu.VMEM((EPC,), jnp.int32),
            pltpu.VMEM((EPC,), jnp.int32),
            pltpu.VMEM((EPC,), jnp.int32),
            pltpu.VMEM((NCHUNK, KCH), jnp.int32),
            pltpu.VMEM((NCHUNK, KCH), jnp.int32),
        ],
    )(_edge_prep_body)


def _edge_prep_body(src_h, dst_h, typ_h, gidx_h, sidx_h,
                    src_v, dst_v, typ_v, gidx_v, sidx_v):
    c = lax.axis_index("c")
    s = lax.axis_index("s")
    base = s * EPC
    pltpu.sync_copy(src_h.at[pl.ds(base, EPC)], src_v)
    pltpu.sync_copy(dst_h.at[pl.ds(base, EPC)], dst_v)
    pltpu.sync_copy(typ_h.at[pl.ds(base, EPC)], typ_v)

    for k in range(KCH // LANES):
        gidx_v[NCHUNK - 1, pl.ds(k * LANES, LANES)] = jnp.zeros((LANES,), jnp.int32)
        sidx_v[NCHUNK - 1, pl.ds(k * LANES, LANES)] = jnp.full((LANES,), TRASH, jnp.int32)

    def _body(i, carry):
        sv = src_v[pl.ds(i * LANES, LANES)]
        dv = dst_v[pl.ds(i * LANES, LANES)]
        tv = typ_v[pl.ds(i * LANES, LANES)]
        keep = tv == c
        row = i // (KCH // LANES)
        col = (i % (KCH // LANES)) * LANES
        gidx_v[row, pl.ds(col, LANES)] = sv
        sidx_v[row, pl.ds(col, LANES)] = jnp.where(keep, dv, TRASH)
        return carry

    lax.fori_loop(0, NVR, _body, 0)
    pltpu.sync_copy(gidx_v, gidx_h.at[c, s])
    pltpu.sync_copy(sidx_v, sidx_h.at[c, s])


@functools.cache
def _make_edge_cnt():
    mesh = plsc.VectorSubcoreMesh(core_axis_name="c", subcore_axis_name="s")
    return functools.partial(
        pl.kernel,
        mesh=mesh,
        out_type=jax.ShapeDtypeStruct((NREL, NPAD, CW), jnp.float32),
        scratch_types=[
            pltpu.VMEM((NCHUNK, KCH), jnp.int32),
            pltpu.VMEM((KCH, CW), jnp.float32),
            pltpu.VMEM_SHARED((NPAD, CW), jnp.float32),
        ],
    )(_edge_cnt_body)


def _edge_cnt_body(sidx_h, cnt_h, sidx_v, val_v, cnt_sh):
    c = lax.axis_index("c")
    s = lax.axis_index("s")
    pltpu.sync_copy(sidx_h.at[c, s], sidx_v)

    zeros16 = jnp.zeros((LANES,), jnp.float32)

    def _zfill(i, carry):
        for k in range(CW // LANES):
            val_v[i, pl.ds(k * LANES, LANES)] = zeros16
        return carry

    lax.fori_loop(0, KCH, _zfill, 0)
    for k in range(RPS // KCH):
        pltpu.sync_copy(val_v, cnt_sh.at[pl.ds(s * RPS + k * KCH, KCH)])

    ones16 = jnp.ones((LANES,), jnp.float32)

    def _ofill(i, carry):
        for k in range(CW // LANES):
            val_v[i, pl.ds(k * LANES, LANES)] = ones16
        return carry

    lax.fori_loop(0, KCH, _ofill, 0)
    plsc.subcore_barrier()

    def _cnt(j, carry):
        pltpu.sync_copy(val_v, cnt_sh.at[sidx_v.at[j]], add=True)
        return carry

    lax.fori_loop(0, NCHUNK, _cnt, 0)
    plsc.subcore_barrier()
    for k in range(RPS // KCH):
        rows = pl.ds(s * RPS + k * KCH, KCH)
        pltpu.sync_copy(cnt_sh.at[rows], val_v)
        pltpu.sync_copy(val_v, cnt_h.at[c, rows])


@functools.cache
def _make_edge_agg():
    mesh = plsc.VectorSubcoreMesh(core_axis_name="c", subcore_axis_name="s")
    return functools.partial(
        pl.kernel,
        mesh=mesh,
        out_type=jax.ShapeDtypeStruct((NREL, NPAD, D), jnp.float32),
        scratch_types=[
            pltpu.VMEM((NCHUNK, KCH), jnp.int32),
            pltpu.VMEM((NCHUNK, KCH), jnp.int32),
            pltpu.VMEM((KCH, D), jnp.float32),
            pltpu.VMEM_SHARED((NPAD, D), jnp.float32),
        ],
    )(_edge_agg_body)


def _edge_agg_body(x_h, gidx_h, sidx_h, agg_h, gidx_v, sidx_v, row_v, agg_sh):
    c = lax.axis_index("c")
    s = lax.axis_index("s")
    pltpu.sync_copy(gidx_h.at[c, s], gidx_v)
    pltpu.sync_copy(sidx_h.at[c, s], sidx_v)

    zeros16 = jnp.zeros((LANES,), jnp.float32)

    def _zfill(i, carry):
        for k in range(D // LANES):
            row_v[i, pl.ds(k * LANES, LANES)] = zeros16
        return carry

    lax.fori_loop(0, KCH, _zfill, 0)
    for k in range(RPS // KCH):
        pltpu.sync_copy(row_v, agg_sh.at[pl.ds(s * RPS + k * KCH, KCH)])
    plsc.subcore_barrier()

    def _body(j, carry):
        pltpu.sync_copy(x_h.at[gidx_v.at[j]], row_v)
        pltpu.sync_copy(row_v, agg_sh.at[sidx_v.at[j]], add=True)
        return carry

    lax.fori_loop(0, NCHUNK, _body, 0)
    plsc.subcore_barrier()
    for k in range(RPS // KCH):
        rows = pl.ds(s * RPS + k * KCH, KCH)
        pltpu.sync_copy(agg_sh.at[rows], row_v)
        pltpu.sync_copy(row_v, agg_h.at[c, rows])


def _conv_bn_body(x_ref, wo_ref, bias_ref, wr_ref, agg_ref, cnt_ref,
                  g_ref, be_ref, o_ref):
    x = x_ref[...]
    acc = jnp.dot(x, wo_ref[...], preferred_element_type=jnp.float32)
    acc = acc + bias_ref[...]
    for r in range(NREL):
        a = agg_ref[r, :N, :]
        cnt = jnp.sum(cnt_ref[r, :N, :], axis=1, keepdims=True) * (1.0 / CW)
        scale = 1.0 / jnp.maximum(cnt, 1.0)
        acc = acc + jnp.dot(a * scale, wr_ref[r],
                            preferred_element_type=jnp.float32)
    mu = jnp.mean(acc, axis=0, keepdims=True)
    dlt = acc - mu
    var = jnp.mean(dlt * dlt, axis=0, keepdims=True)
    y = dlt * (g_ref[...] * lax.rsqrt(var + 1e-5)) + be_ref[...]
    o_ref[...] = jnp.where(y >= 0, y, 0.01 * y)


def _conv_bn(x, wo, bias, wr, agg, cnt, g, be):
    return pl.pallas_call(
        _conv_bn_body,
        out_shape=jax.ShapeDtypeStruct((N, D), jnp.float32),
    )(x, wo, bias.reshape(1, D), wr, agg, cnt,
      g.reshape(1, D), be.reshape(1, D))


def _pool_body(h_ref, b_ref, o_ref):
    h = h_ref[...]
    bb = b_ref[...]
    cols = lax.broadcasted_iota(jnp.int32, (N, D), 1)
    oh = (bb == cols).astype(jnp.float32)
    sums = lax.dot_general(oh, h, (((0,), (0,)), ((), ())),
                           preferred_element_type=jnp.float32)
    cnts = jnp.sum(oh, axis=0)[:, None]
    o_ref[...] = (sums / jnp.maximum(cnts, 1.0))[:NB, :]


def _pool(h, batch):
    return pl.pallas_call(
        _pool_body,
        out_shape=jax.ShapeDtypeStruct((NB, D), jnp.float32),
    )(h, batch.reshape(N, 1))


def _branch(x, ei, et, batch, Wr1, Wo1, bb1, g1, be1, Wr2, Wo2, bb2, g2, be2):
    ei = ei.astype(jnp.int32)
    et = et.astype(jnp.int32)
    src, dst = ei[0], ei[1]
    gidx, sidx = _make_edge_prep()(src, dst, et)
    cnt = _make_edge_cnt()(sidx)
    agg1 = _make_edge_agg()(x, gidx, sidx)
    h1 = _conv_bn(x, Wo1, bb1, Wr1, agg1, cnt, g1, be1)
    agg2 = _make_edge_agg()(h1, gidx, sidx)
    h2 = _conv_bn(h1, Wo2, bb2, Wr2, agg2, cnt, g2, be2)
    return _pool(h2, batch.astype(jnp.int32))


def kernel(x_t, edge_index_t, edge_type_t, x_t_batch,
           x_p, edge_index_p, edge_type_p, x_p_batch,
           t_Wr1, t_Wo1, t_bb1, t_g1, t_be1, t_Wr2, t_Wo2, t_bb2, t_g2, t_be2,
           p_Wr1, p_Wo1, p_bb1, p_g1, p_be1, p_Wr2, p_Wo2, p_bb2, p_g2, p_be2):
    out_t = _branch(x_t, edge_index_t, edge_type_t, x_t_batch,
                    t_Wr1, t_Wo1, t_bb1, t_g1, t_be1,
                    t_Wr2, t_Wo2, t_bb2, t_g2, t_be2)
    out_p = _branch(x_p, edge_index_p, edge_type_p, x_p_batch,
                    p_Wr1, p_Wo1, p_bb1, p_g1, p_be1,
                    p_Wr2, p_Wo2, p_bb2, p_g2, p_be2)
    return (out_t, out_p)
